# baseline (speedup 1.0000x reference)
import jax
import jax.numpy as jnp
from jax import lax
from jax.experimental import pallas as pl
from jax.experimental.pallas import tpu as pltpu
from jax.experimental.pallas import tpu_sc as plsc

_D = 64
_B = 4096
_S = 200
_NW = 32
_NBLK = _S
_GBUF = 4
_TBUF = 2
_SCALE = 8.0


def _body(idx_hbm, tab_hbm, out_hbm, idx_all, rows, tbl, sems_g, sems_o):
    c = lax.axis_index("c")
    s = lax.axis_index("s")
    w = s * 2 + c

    pltpu.sync_copy(idx_hbm.at[pl.ds(0, _S), pl.ds(w * 128, 128)], idx_all)

    iota = lax.iota(jnp.int32, 16)
    zeros16 = jnp.zeros((16,), jnp.int32)
    dvs = [iota + 16 * cc for cc in range(4)]

    def issue_gather(k, g):
        pltpu.async_copy(tab_hbm.at[idx_all.at[k]], rows.at[g], sems_g[g])

    def wait_gather(g):
        pltpu.make_async_copy(
            tab_hbm.at[pl.ds(0, 128)], rows.at[g], sems_g[g]
        ).wait()

    def out_slice(k):
        return out_hbm.at[pl.ds(k, 1), pl.ds(0, _D), pl.ds(w * 128, 128)]

    def issue_out(k, p):
        pltpu.async_copy(tbl.at[p], out_slice(k), sems_o[p])

    def wait_out(k, p):
        pltpu.make_async_copy(tbl.at[p], out_slice(k), sems_o[p]).wait()

    issue_gather(0, 0)
    issue_gather(1, 1)

    def group(k4, carry):
        for j in range(_GBUF):
            k = k4 * _GBUF + j
            g = j
            p = j % _TBUF

            @pl.when(k >= _TBUF)
            def _():
                wait_out(k - _TBUF, p)

            @pl.when(k + 2 < _NBLK)
            def _():
                issue_gather(k + 2, (j + 2) % _GBUF)

            wait_gather(g)

            def row(b, rcarry):
                bspl = zeros16 + b
                for cc in range(4):
                    v = rows[g, b, pl.ds(cc * 16, 16)] * _SCALE
                    plsc.store_scatter(
                        tbl.at[p, 0], [dvs[cc], bspl], v
                    )
                return rcarry

            lax.fori_loop(0, 128, row, 0, unroll=8)
            issue_out(k, p)
        return carry

    lax.fori_loop(0, _NBLK // _GBUF, group, 0)

    wait_out(_NBLK - 2, (_NBLK - 2) % _TBUF)
    wait_out(_NBLK - 1, (_NBLK - 1) % _TBUF)


def kernel(vocab_to_embed, embedding_table):
    idxT = vocab_to_embed.T.astype(jnp.int32)
    tab128 = jnp.pad(embedding_table, ((0, 0), (0, _D)))
    mesh = plsc.VectorSubcoreMesh(core_axis_name="c", subcore_axis_name="s")
    f = pl.kernel(
        _body,
        mesh=mesh,
        compiler_params=pltpu.CompilerParams(
            use_tc_tiling_on_sc=True, needs_layout_passes=False
        ),
        out_type=jax.ShapeDtypeStruct((_S, _D, _B), jnp.float32),
        scratch_types=[
            pltpu.VMEM((_S, 128), jnp.int32),
            pltpu.VMEM((_GBUF, 128, 128), jnp.float32),
            pltpu.VMEM((_TBUF, 1, _D, 128), jnp.float32),
            [pltpu.SemaphoreType.DMA] * _GBUF,
            [pltpu.SemaphoreType.DMA] * _TBUF,
        ],
    )
    out3 = f(idxT, tab128)
    return out3.transpose(2, 0, 1)


# speedup vs baseline: 1.3314x; 1.3314x over previous
import jax
import jax.numpy as jnp
from jax import lax
from jax.experimental import pallas as pl
from jax.experimental.pallas import tpu as pltpu
from jax.experimental.pallas import tpu_sc as plsc

_D = 64
_B = 4096
_S = 200
_NW = 32
_NBLK = _S
_GBUF = 4
_TBUF = 2
_SCALE = 8.0


def _body(idx_hbm, tab_hbm, out_hbm, idx_all, rows, tbl, sems_g, sems_o):
    c = lax.axis_index("c")
    s = lax.axis_index("s")
    w = s * 2 + c

    pltpu.sync_copy(idx_hbm.at[pl.ds(0, _S), pl.ds(w * 128, 128)], idx_all)

    iota = lax.iota(jnp.int32, 16)
    zeros16 = jnp.zeros((16,), jnp.int32)
    dvs = [iota + 16 * cc for cc in range(4)]

    def issue_gather(k, g):
        pltpu.async_copy(tab_hbm.at[idx_all.at[k]], rows.at[g], sems_g[g])

    def wait_gather(g):
        pltpu.make_async_copy(
            tab_hbm.at[pl.ds(0, 128)], rows.at[g], sems_g[g]
        ).wait()

    def out_slice(k):
        return out_hbm.at[pl.ds(k, 1), pl.ds(0, _D), pl.ds(w * 128, 128)]

    def issue_out(k, p):
        pltpu.async_copy(tbl.at[p], out_slice(k), sems_o[p])

    def wait_out(k, p):
        pltpu.make_async_copy(tbl.at[p], out_slice(k), sems_o[p]).wait()

    issue_gather(0, 0)
    issue_gather(1, 1)

    def group(k4, carry):
        for j in range(_GBUF):
            k = k4 * _GBUF + j
            g = j
            p = j % _TBUF

            @pl.when(k >= _TBUF)
            def _():
                wait_out(k - _TBUF, p)

            @pl.when(k + 2 < _NBLK)
            def _():
                issue_gather(k + 2, (j + 2) % _GBUF)

            wait_gather(g)

            @plsc.parallel_loop(0, 128, unroll=8)
            def _row(b):
                bspl = zeros16 + b
                for cc in range(4):
                    v = rows[g, b, pl.ds(cc * 16, 16)] * _SCALE
                    plsc.store_scatter(
                        tbl.at[p, 0], [dvs[cc], bspl], v
                    )
            issue_out(k, p)
        return carry

    lax.fori_loop(0, _NBLK // _GBUF, group, 0)

    wait_out(_NBLK - 2, (_NBLK - 2) % _TBUF)
    wait_out(_NBLK - 1, (_NBLK - 1) % _TBUF)


def kernel(vocab_to_embed, embedding_table):
    idxT = vocab_to_embed.T.astype(jnp.int32)
    tab128 = jnp.pad(embedding_table, ((0, 0), (0, _D)))
    mesh = plsc.VectorSubcoreMesh(core_axis_name="c", subcore_axis_name="s")
    f = pl.kernel(
        _body,
        mesh=mesh,
        compiler_params=pltpu.CompilerParams(
            use_tc_tiling_on_sc=True, needs_layout_passes=False
        ),
        out_type=jax.ShapeDtypeStruct((_S, _D, _B), jnp.float32),
        scratch_types=[
            pltpu.VMEM((_S, 128), jnp.int32),
            pltpu.VMEM((_GBUF, 128, 128), jnp.float32),
            pltpu.VMEM((_TBUF, 1, _D, 128), jnp.float32),
            [pltpu.SemaphoreType.DMA] * _GBUF,
            [pltpu.SemaphoreType.DMA] * _TBUF,
        ],
    )
    out3 = f(idxT, tab128)
    return out3.transpose(2, 0, 1)


# speedup vs baseline: 2.1830x; 1.6397x over previous
import jax
import jax.numpy as jnp
from jax import lax
from jax.experimental import pallas as pl
from jax.experimental.pallas import tpu as pltpu
from jax.experimental.pallas import tpu_sc as plsc

_D = 64
_B = 4096
_S = 200
_NW = 32
_NBLK = _S
_GBUF = 4
_TBUF = 2
_SCALE = 8.0


def _body(idx_hbm, tab_hbm, out_hbm, idx_all, rows, tbl, sems_g, sems_o):
    c = lax.axis_index("c")
    s = lax.axis_index("s")
    w = s * 2 + c

    pltpu.sync_copy(idx_hbm.at[pl.ds(0, _S), pl.ds(w * 128, 128)], idx_all)

    iota = lax.iota(jnp.int32, 16)
    zeros16 = jnp.zeros((16,), jnp.int32)
    dvs = [iota + 16 * cc for cc in range(4)]

    def issue_gather(k, g):
        pltpu.async_copy(tab_hbm.at[idx_all.at[k]], rows.at[g], sems_g[g])

    def wait_gather(g):
        pltpu.make_async_copy(
            tab_hbm.at[pl.ds(0, 128)], rows.at[g], sems_g[g]
        ).wait()

    def out_slice(k):
        return out_hbm.at[pl.ds(k, 1), pl.ds(0, _D), pl.ds(w * 128, 128)]

    def issue_out(k, p):
        pltpu.async_copy(tbl.at[p], out_slice(k), sems_o[p])

    def wait_out(k, p):
        pltpu.make_async_copy(tbl.at[p], out_slice(k), sems_o[p]).wait()

    issue_gather(0, 0)
    issue_gather(1, 1)

    def group(k4, carry):
        for j in range(_GBUF):
            k = k4 * _GBUF + j
            g = j
            p = j % _TBUF

            @pl.when(k >= _TBUF)
            def _():
                wait_out(k - _TBUF, p)

            @pl.when(k + 2 < _NBLK)
            def _():
                issue_gather(k + 2, (j + 2) % _GBUF)

            wait_gather(g)

            @plsc.parallel_loop(0, 128, unroll=4)
            def _diag(it):
                b0 = (it // 16) * 16
                r = it % 16
                brot = b0 + ((iota + r) & 15)
                for cc in range(4):
                    v = plsc.load_gather(rows.at[g], [brot, dvs[cc]]) * _SCALE
                    plsc.store_scatter(tbl.at[p, 0], [dvs[cc], brot], v)
            issue_out(k, p)
        return carry

    lax.fori_loop(0, _NBLK // _GBUF, group, 0)

    wait_out(_NBLK - 2, (_NBLK - 2) % _TBUF)
    wait_out(_NBLK - 1, (_NBLK - 1) % _TBUF)


def kernel(vocab_to_embed, embedding_table):
    idxT = vocab_to_embed.T.astype(jnp.int32)
    tab128 = jnp.pad(embedding_table, ((0, 0), (0, _D)))
    mesh = plsc.VectorSubcoreMesh(core_axis_name="c", subcore_axis_name="s")
    f = pl.kernel(
        _body,
        mesh=mesh,
        compiler_params=pltpu.CompilerParams(
            use_tc_tiling_on_sc=True, needs_layout_passes=False
        ),
        out_type=jax.ShapeDtypeStruct((_S, _D, _B), jnp.float32),
        scratch_types=[
            pltpu.VMEM((_S, 128), jnp.int32),
            pltpu.VMEM((_GBUF, 128, 128), jnp.float32),
            pltpu.VMEM((_TBUF, 1, _D, 128), jnp.float32),
            [pltpu.SemaphoreType.DMA] * _GBUF,
            [pltpu.SemaphoreType.DMA] * _TBUF,
        ],
    )
    out3 = f(idxT, tab128)
    return out3.transpose(2, 0, 1)


# speedup vs baseline: 3.1254x; 1.4317x over previous
import jax
import jax.numpy as jnp
from jax import lax
from jax.experimental import pallas as pl
from jax.experimental.pallas import tpu as pltpu
from jax.experimental.pallas import tpu_sc as plsc

_D = 64
_B = 4096
_S = 200
_NW = 32
_NBLK = _S
_GBUF = 4
_TBUF = 2
_SCALE = 8.0


_TCH = 244


def _tbody(tabT_hbm, tail_hbm, out_hbm, tin, tout, sems_i, sems_to):
    c = lax.axis_index("c")
    s = lax.axis_index("s")
    w = s * 2 + c
    base = w * (_TCH * 128)

    iota = lax.iota(jnp.int32, 16)
    dvs = [iota + 16 * cc for cc in range(4)]

    def issue_in(k, b):
        pltpu.async_copy(
            tabT_hbm.at[pl.ds(0, _D), pl.ds(base + k * 128, 128)],
            tin.at[b],
            sems_i[b],
        )

    def wait_in(b):
        pltpu.make_async_copy(
            tabT_hbm.at[pl.ds(0, _D), pl.ds(0, 128)], tin.at[b], sems_i[b]
        ).wait()

    def issue_out(k, b):
        pltpu.async_copy(
            tout.at[b], out_hbm.at[pl.ds(base + k * 128, 128)], sems_to[b]
        )

    def wait_out(k, b):
        pltpu.make_async_copy(
            tout.at[b], out_hbm.at[pl.ds(base + k * 128, 128)], sems_to[b]
        ).wait()

    issue_in(0, 0)
    issue_in(1, 1)

    def group(k2, carry):
        for j in range(2):
            k = k2 * 2 + j
            b = j
            wait_in(b)

            @pl.when(k >= 2)
            def _():
                wait_out(k - 2, b)

            diag_transpose(b, 128)
            issue_out(k, b)

            @pl.when(k + 2 < _TCH)
            def _():
                issue_in(k + 2, b)
        return carry

    def diag_transpose(b, nv):
        @plsc.parallel_loop(0, nv, unroll=4)
        def _diag(it):
            v0 = (it // 16) * 16
            r = it % 16
            vrot = v0 + ((iota + r) & 15)
            for cc in range(4):
                x = plsc.load_gather(tin.at[b], [dvs[cc], vrot])
                plsc.store_scatter(tout.at[b], [vrot, dvs[cc]], x)

    lax.fori_loop(0, _TCH // 2, group, 0)

    wait_out(_TCH - 2, 0)
    wait_out(_TCH - 1, 1)

    for t in range(4):
        v0t = 999424 + t * 128
        pltpu.sync_copy(tabT_hbm.at[pl.ds(0, _D), pl.ds(v0t, 128)], tin.at[0])
        diag_transpose(0, 128)
        pltpu.sync_copy(tout.at[0], out_hbm.at[pl.ds(v0t, 128)])
    pltpu.sync_copy(tail_hbm, tin.at[0])
    pltpu.sync_copy(tin.at[0], out_hbm.at[pl.ds(999936, 64)])


def _body(idx_hbm, tab_hbm, out_hbm, idx_all, rows, tbl, sems_g, sems_o):
    c = lax.axis_index("c")
    s = lax.axis_index("s")
    w = s * 2 + c

    pltpu.sync_copy(idx_hbm.at[pl.ds(0, _S), pl.ds(w * 128, 128)], idx_all)

    iota = lax.iota(jnp.int32, 16)
    zeros16 = jnp.zeros((16,), jnp.int32)
    dvs = [iota + 16 * cc for cc in range(4)]

    def issue_gather(k, g):
        pltpu.async_copy(tab_hbm.at[idx_all.at[k]], rows.at[g], sems_g[g])

    def wait_gather(g):
        pltpu.make_async_copy(
            tab_hbm.at[pl.ds(0, 128)], rows.at[g], sems_g[g]
        ).wait()

    def out_slice(k):
        return out_hbm.at[pl.ds(k, 1), pl.ds(0, _D), pl.ds(w * 128, 128)]

    def issue_out(k, p):
        pltpu.async_copy(tbl.at[p], out_slice(k), sems_o[p])

    def wait_out(k, p):
        pltpu.make_async_copy(tbl.at[p], out_slice(k), sems_o[p]).wait()

    issue_gather(0, 0)
    issue_gather(1, 1)

    def group(k4, carry):
        for j in range(_GBUF):
            k = k4 * _GBUF + j
            g = j
            p = j % _TBUF

            @pl.when(k >= _TBUF)
            def _():
                wait_out(k - _TBUF, p)

            @pl.when(k + 2 < _NBLK)
            def _():
                issue_gather(k + 2, (j + 2) % _GBUF)

            wait_gather(g)

            @plsc.parallel_loop(0, 128, unroll=4)
            def _diag(it):
                b0 = (it // 16) * 16
                r = it % 16
                brot = b0 + ((iota + r) & 15)
                for cc in range(4):
                    v = plsc.load_gather(rows.at[g], [brot, dvs[cc]]) * _SCALE
                    plsc.store_scatter(tbl.at[p, 0], [dvs[cc], brot], v)
            issue_out(k, p)
        return carry

    lax.fori_loop(0, _NBLK // _GBUF, group, 0)

    wait_out(_NBLK - 2, (_NBLK - 2) % _TBUF)
    wait_out(_NBLK - 1, (_NBLK - 1) % _TBUF)


def kernel(vocab_to_embed, embedding_table):
    idxT = vocab_to_embed.T.astype(jnp.int32)
    mesh = plsc.VectorSubcoreMesh(core_axis_name="c", subcore_axis_name="s")
    tf = pl.kernel(
        _tbody,
        mesh=mesh,
        compiler_params=pltpu.CompilerParams(
            use_tc_tiling_on_sc=True, needs_layout_passes=False
        ),
        out_type=jax.ShapeDtypeStruct((1000000, 128), jnp.float32),
        scratch_types=[
            pltpu.VMEM((2, _D, 128), jnp.float32),
            pltpu.VMEM((2, 128, 128), jnp.float32),
            [pltpu.SemaphoreType.DMA] * 2,
            [pltpu.SemaphoreType.DMA] * 2,
        ],
    )
    tail128 = jnp.pad(embedding_table[999936:], ((0, 0), (0, _D)))
    tab128 = tf(embedding_table.T, tail128)
    f = pl.kernel(
        _body,
        mesh=mesh,
        compiler_params=pltpu.CompilerParams(
            use_tc_tiling_on_sc=True, needs_layout_passes=False
        ),
        out_type=jax.ShapeDtypeStruct((_S, _D, _B), jnp.float32),
        scratch_types=[
            pltpu.VMEM((_S, 128), jnp.int32),
            pltpu.VMEM((_GBUF, 128, 128), jnp.float32),
            pltpu.VMEM((_TBUF, 1, _D, 128), jnp.float32),
            [pltpu.SemaphoreType.DMA] * _GBUF,
            [pltpu.SemaphoreType.DMA] * _TBUF,
        ],
    )
    out3 = f(idxT, tab128)
    return out3.transpose(2, 0, 1)


# speedup vs baseline: 3.5232x; 1.1273x over previous
import jax
import jax.numpy as jnp
from jax import lax
from jax.experimental import pallas as pl
from jax.experimental.pallas import tpu as pltpu
from jax.experimental.pallas import tpu_sc as plsc

_D = 64
_B = 4096
_S = 200
_NW = 32
_NBLK = _S
_GBUF = 4
_TBUF = 2
_SCALE = 8.0


_TCH = 244


def _tbody(tabT_hbm, tail_hbm, out_hbm, tin, tout, sems_i, sems_to):
    c = lax.axis_index("c")
    s = lax.axis_index("s")
    w = s * 2 + c
    base = w * (_TCH * 128)

    iota = lax.iota(jnp.int32, 16)
    dvs = [iota + 16 * cc for cc in range(4)]

    def issue_in(k, b):
        pltpu.async_copy(
            tabT_hbm.at[pl.ds(0, _D), pl.ds(base + k * 128, 128)],
            tin.at[b],
            sems_i[b],
        )

    def wait_in(b):
        pltpu.make_async_copy(
            tabT_hbm.at[pl.ds(0, _D), pl.ds(0, 128)], tin.at[b], sems_i[b]
        ).wait()

    def issue_out(k, b):
        off = pl.multiple_of(w * (_TCH * 64) + k * 64, 64)
        pltpu.async_copy(tout.at[b], out_hbm.at[pl.ds(off, 64)], sems_to[b])

    def wait_out(k, b):
        off = pl.multiple_of(w * (_TCH * 64) + k * 64, 64)
        pltpu.make_async_copy(
            tout.at[b], out_hbm.at[pl.ds(off, 64)], sems_to[b]
        ).wait()

    issue_in(0, 0)
    issue_in(1, 1)

    def group(k2, carry):
        for j in range(2):
            k = k2 * 2 + j
            b = j
            wait_in(b)

            @pl.when(k >= 2)
            def _():
                wait_out(k - 2, b)

            diag_transpose(b, 128)
            issue_out(k, b)

            @pl.when(k + 2 < _TCH)
            def _():
                issue_in(k + 2, b)
        return carry

    def diag_transpose(b, nv):
        @plsc.parallel_loop(0, nv, unroll=4)
        def _diag(it):
            v0 = (it // 16) * 16
            r = it % 16
            vrot = v0 + ((iota + r) & 15)
            vhalf = vrot >> 1
            voff = (vrot & 1) << 6
            for cc in range(4):
                x = plsc.load_gather(tin.at[b], [dvs[cc], vrot])
                plsc.store_scatter(tout.at[b], [vhalf, voff + dvs[cc]], x)

    lax.fori_loop(0, _TCH // 2, group, 0)

    wait_out(_TCH - 2, 0)
    wait_out(_TCH - 1, 1)

    for t in range(4):
        v0t = 999424 + t * 128
        pltpu.sync_copy(tabT_hbm.at[pl.ds(0, _D), pl.ds(v0t, 128)], tin.at[0])
        diag_transpose(0, 128)
        pltpu.sync_copy(tout.at[0], out_hbm.at[pl.ds(v0t // 2, 64)])
    pltpu.sync_copy(tail_hbm, tin.at[0, pl.ds(0, 32)])
    pltpu.sync_copy(tin.at[0, pl.ds(0, 32)], out_hbm.at[pl.ds(499968, 32)])


def _body(idx_hbm, tab_hbm, out_hbm, idx_all, rows, tbl, pairb, parb,
          sems_g, sems_o):
    c = lax.axis_index("c")
    s = lax.axis_index("s")
    w = s * 2 + c

    pltpu.sync_copy(idx_hbm.at[pl.ds(0, _S), pl.ds(w * 128, 128)], idx_all)

    iota = lax.iota(jnp.int32, 16)
    zeros16 = jnp.zeros((16,), jnp.int32)
    dvs = [iota + 16 * cc for cc in range(4)]

    def issue_gather(k, g):
        @plsc.parallel_loop(0, 8)
        def _pair(q):
            pairb[g, pl.ds(q * 16, 16)] = idx_all[k, pl.ds(q * 16, 16)] >> 1

        pltpu.async_copy(tab_hbm.at[pairb.at[g]], rows.at[g], sems_g[g])

    def wait_gather(g):
        pltpu.make_async_copy(
            tab_hbm.at[pl.ds(0, 128)], rows.at[g], sems_g[g]
        ).wait()

    def out_slice(k):
        return out_hbm.at[pl.ds(k, 1), pl.ds(0, _D), pl.ds(w * 128, 128)]

    def issue_out(k, p):
        pltpu.async_copy(tbl.at[p], out_slice(k), sems_o[p])

    def wait_out(k, p):
        pltpu.make_async_copy(tbl.at[p], out_slice(k), sems_o[p]).wait()

    issue_gather(0, 0)
    issue_gather(1, 1)

    def group(k4, carry):
        for j in range(_GBUF):
            k = k4 * _GBUF + j
            g = j
            p = j % _TBUF

            @pl.when(k >= _TBUF)
            def _():
                wait_out(k - _TBUF, p)

            @pl.when(k + 2 < _NBLK)
            def _():
                issue_gather(k + 2, (j + 2) % _GBUF)

            wait_gather(g)

            @plsc.parallel_loop(0, 8)
            def _par(q):
                parb[pl.ds(q * 16, 16)] = (
                    idx_all[k, pl.ds(q * 16, 16)] & 1
                ) << 6

            @plsc.parallel_loop(0, 128, unroll=4)
            def _diag(it):
                b0 = (it // 16) * 16
                r = it % 16
                brot = b0 + ((iota + r) & 15)
                par = plsc.load_gather(parb, [brot])
                for cc in range(4):
                    v = plsc.load_gather(rows.at[g], [brot, par + dvs[cc]])
                    plsc.store_scatter(tbl.at[p, 0], [dvs[cc], brot], v * _SCALE)
            issue_out(k, p)
        return carry

    lax.fori_loop(0, _NBLK // _GBUF, group, 0)

    wait_out(_NBLK - 2, (_NBLK - 2) % _TBUF)
    wait_out(_NBLK - 1, (_NBLK - 1) % _TBUF)


def kernel(vocab_to_embed, embedding_table):
    idxT = vocab_to_embed.T.astype(jnp.int32)
    mesh = plsc.VectorSubcoreMesh(core_axis_name="c", subcore_axis_name="s")
    tf = pl.kernel(
        _tbody,
        mesh=mesh,
        compiler_params=pltpu.CompilerParams(
            use_tc_tiling_on_sc=True, needs_layout_passes=False
        ),
        out_type=jax.ShapeDtypeStruct((500000, 128), jnp.float32),
        scratch_types=[
            pltpu.VMEM((2, _D, 128), jnp.float32),
            pltpu.VMEM((2, _D, 128), jnp.float32),
            [pltpu.SemaphoreType.DMA] * 2,
            [pltpu.SemaphoreType.DMA] * 2,
        ],
    )
    tail128 = embedding_table[999936:].reshape(32, 128)
    tab128 = tf(embedding_table.T, tail128)
    f = pl.kernel(
        _body,
        mesh=mesh,
        compiler_params=pltpu.CompilerParams(
            use_tc_tiling_on_sc=True, needs_layout_passes=False
        ),
        out_type=jax.ShapeDtypeStruct((_S, _D, _B), jnp.float32),
        scratch_types=[
            pltpu.VMEM((_S, 128), jnp.int32),
            pltpu.VMEM((_GBUF, 128, 128), jnp.float32),
            pltpu.VMEM((_TBUF, 1, _D, 128), jnp.float32),
            pltpu.VMEM((_GBUF, 128), jnp.int32),
            pltpu.VMEM((128,), jnp.int32),
            [pltpu.SemaphoreType.DMA] * _GBUF,
            [pltpu.SemaphoreType.DMA] * _TBUF,
        ],
    )
    out3 = f(idxT, tab128)
    return out3.transpose(2, 0, 1)
